# baseline (speedup 1.0000x reference)
import functools
import math

import jax
import jax.numpy as jnp
from jax import lax
from jax.experimental import pallas as pl
from jax.experimental.pallas import tpu as pltpu
from jax.experimental.pallas import tpu_sc as plsc

D_MODEL = 64
SCALE = math.sqrt(D_MODEL)
CHUNK = 100
REPACK_BLK = 1000


def _repack_body(a_ref, b_ref, o_ref):
    o_ref[:, 0:D_MODEL] = a_ref[...] * SCALE
    o_ref[:, D_MODEL:2 * D_MODEL] = b_ref[...] * SCALE


@functools.cache
def _build_repack(V):
    H = V // 2
    nblk = H // REPACK_BLK
    return pl.pallas_call(
        _repack_body,
        grid=(nblk,),
        in_specs=[
            pl.BlockSpec((REPACK_BLK, D_MODEL), lambda j: (j, 0)),
            pl.BlockSpec((REPACK_BLK, D_MODEL), lambda j, n=nblk: (j + n, 0)),
        ],
        out_specs=pl.BlockSpec((REPACK_BLK, 2 * D_MODEL), lambda j: (j, 0)),
        out_shape=jax.ShapeDtypeStruct((H, 2 * D_MODEL), jnp.float32),
    )


@functools.cache
def _build_gather(R, C, V):
    info = plsc.get_sparse_core_info()
    nc, ns = info.num_cores, info.num_subcores
    nw = nc * ns
    B = R * C
    b_per_w = B // nw
    rows_per_w = R // nw
    cpr = C // CHUNK
    n_chunks = b_per_w // CHUNK
    NBUF = 4
    assert n_chunks % NBUF == 0 and n_chunks >= 3 * NBUF
    mesh = plsc.VectorSubcoreMesh(core_axis_name="c", subcore_axis_name="s")

    @functools.partial(
        pl.kernel,
        mesh=mesh,
        compiler_params=pltpu.CompilerParams(use_tc_tiling_on_sc=False),
        out_type=jax.ShapeDtypeStruct((R, C, D_MODEL), jnp.float32),
        scratch_types=[
            pltpu.VMEM((n_chunks, CHUNK), jnp.int32),
            pltpu.VMEM((CHUNK, D_MODEL), jnp.float32),
            pltpu.VMEM((CHUNK, D_MODEL), jnp.float32),
            pltpu.VMEM((CHUNK, D_MODEL), jnp.float32),
            pltpu.VMEM((CHUNK, D_MODEL), jnp.float32),
            pltpu.SemaphoreType.DMA,
            pltpu.SemaphoreType.DMA,
            pltpu.SemaphoreType.DMA,
            pltpu.SemaphoreType.DMA,
            pltpu.SemaphoreType.DMA,
            pltpu.SemaphoreType.DMA,
            pltpu.SemaphoreType.DMA,
            pltpu.SemaphoreType.DMA,
        ],
    )
    def emb_kernel(q_hbm, lut_hbm, out_hbm, q_v, buf0, buf1, buf2, buf3,
                   gsem0, gsem1, gsem2, gsem3, wsem0, wsem1, wsem2, wsem3):
        wid = lax.axis_index("s") * nc + lax.axis_index("c")
        row_base = wid * rows_per_w
        pltpu.sync_copy(q_hbm.at[pl.ds(wid * n_chunks, n_chunks)], q_v)

        bufs = (buf0, buf1, buf2, buf3)
        gsems = (gsem0, gsem1, gsem2, gsem3)
        wsems = (wsem0, wsem1, wsem2, wsem3)

        def gather_start(c, b):
            pltpu.async_copy(lut_hbm.at[q_v.at[c]], bufs[b], gsems[b])

        def gather_wait(b):
            pltpu.make_async_copy(lut_hbm.at[q_v.at[0]], bufs[b],
                                  gsems[b]).wait()

        def write_start(c, b):
            a = row_base + c // cpr
            m0 = (c % cpr) * CHUNK
            pltpu.async_copy(bufs[b], out_hbm.at[a, pl.ds(m0, CHUNK)],
                             wsems[b])

        def write_wait(b):
            pltpu.make_async_copy(bufs[b], out_hbm.at[0, pl.ds(0, CHUNK)],
                                  wsems[b]).wait()

        gather_start(0, 0)
        gather_start(1, 1)
        gather_start(2, 2)
        gather_wait(0)
        write_start(0, 0)
        gather_start(3, 3)
        gather_wait(1)
        write_start(1, 1)
        write_wait(0)
        gather_start(4, 0)
        gather_wait(2)
        write_start(2, 2)
        write_wait(1)
        gather_start(5, 1)
        gather_wait(3)
        write_start(3, 3)

        def quad_body(t, _):
            c = 4 * t
            for k in range(4):
                b = k % NBUF
                pb = (k + 2) % NBUF
                write_wait(pb)
                gather_start(c + k + 2, pb)
                gather_wait(b)
                write_start(c + k, b)
            return ()

        lax.fori_loop(1, n_chunks // 4 - 1, quad_body, ())

        c = n_chunks - 4
        write_wait(2)
        gather_start(c + 2, 2)
        gather_wait(0)
        write_start(c, 0)
        write_wait(3)
        gather_start(c + 3, 3)
        gather_wait(1)
        write_start(c + 1, 1)
        gather_wait(2)
        write_start(c + 2, 2)
        gather_wait(3)
        write_start(c + 3, 3)
        for b in range(NBUF):
            write_wait(b)

    return emb_kernel


def kernel(x, lut):
    V = lut.shape[0]
    H = V // 2
    lut_packed = _build_repack(V)(lut, lut)
    lut_lin = lut_packed.reshape(V, D_MODEL)
    xi = x.reshape(-1).astype(jnp.int32)
    hi = (xi >= H).astype(jnp.int32)
    q = 2 * (xi - H * hi) + hi
    q2 = q.reshape(q.shape[0] // CHUNK, CHUNK)
    return _build_gather(x.shape[0], x.shape[1], V)(q2, lut_lin)


# speedup vs baseline: 1.0805x; 1.0805x over previous
import functools
import math

import jax
import jax.numpy as jnp
from jax import lax
from jax.experimental import pallas as pl
from jax.experimental.pallas import tpu as pltpu
from jax.experimental.pallas import tpu_sc as plsc

D_MODEL = 64
SCALE = math.sqrt(D_MODEL)
CHUNK = 128


@functools.cache
def _build_gather(B, V):
    info = plsc.get_sparse_core_info()
    nc, ns = info.num_cores, info.num_subcores
    nw = nc * ns
    b_per_w = B // nw
    n_chunks = b_per_w // CHUNK
    assert n_chunks % 2 == 0 and n_chunks >= 4
    mesh = plsc.VectorSubcoreMesh(core_axis_name="c", subcore_axis_name="s")

    @functools.partial(
        pl.kernel,
        mesh=mesh,
        compiler_params=pltpu.CompilerParams(use_tc_tiling_on_sc=False),
        out_type=jax.ShapeDtypeStruct((B, D_MODEL), jnp.float32),
        scratch_types=[
            pltpu.VMEM((n_chunks, CHUNK), jnp.int32),
            pltpu.VMEM((CHUNK, D_MODEL), jnp.float32),
            pltpu.VMEM((CHUNK, D_MODEL), jnp.float32),
            pltpu.SemaphoreType.DMA,
            pltpu.SemaphoreType.DMA,
            pltpu.SemaphoreType.DMA,
        ],
    )
    def emb_kernel(idx_hbm, lut_hbm, out_hbm, idx_v, buf0, buf1,
                   gsem0, gsem1, wsem):
        wid = lax.axis_index("s") * nc + lax.axis_index("c")
        base = wid * b_per_w
        pltpu.sync_copy(idx_hbm.at[pl.ds(wid * n_chunks, n_chunks)], idx_v)

        bufs = (buf0, buf1)
        gsems = (gsem0, gsem1)

        def gather_start(c, b):
            pltpu.async_copy(lut_hbm.at[idx_v.at[c]], bufs[b], gsems[b])

        def gather_wait(b):
            pltpu.make_async_copy(lut_hbm.at[idx_v.at[0]], bufs[b],
                                  gsems[b]).wait()

        def scale_buf(b):
            buf = bufs[b]

            def rbody(r, carry):
                for j in range(4):
                    sl = pl.ds(16 * j, 16)
                    buf[r, sl] = buf[r, sl] * SCALE
                return carry

            lax.fori_loop(0, CHUNK, rbody, 0)

        def write_chunk(c, b):
            pltpu.async_copy(bufs[b],
                             out_hbm.at[pl.ds(base + c * CHUNK, CHUNK)],
                             wsem).wait()

        gather_start(0, 0)

        def body(t, carry):
            c = 2 * t
            gather_wait(0)
            gather_start(c + 1, 1)
            scale_buf(0)
            write_chunk(c, 0)
            gather_wait(1)
            gather_start(c + 2, 0)
            scale_buf(1)
            write_chunk(c + 1, 1)
            return carry

        lax.fori_loop(0, n_chunks // 2 - 1, body, 0)

        c = n_chunks - 2
        gather_wait(0)
        gather_start(c + 1, 1)
        scale_buf(0)
        write_chunk(c, 0)
        gather_wait(1)
        scale_buf(1)
        write_chunk(c + 1, 1)

    return emb_kernel


def kernel(x, lut):
    R, C = x.shape
    B = R * C
    xi = x.reshape(-1).astype(jnp.int32)
    idx2 = xi.reshape(B // CHUNK, CHUNK)
    out = _build_gather(B, lut.shape[0])(idx2, lut)
    return out.reshape(R, C, D_MODEL)


# speedup vs baseline: 1.1066x; 1.0242x over previous
import functools
import math

import jax
import jax.numpy as jnp
from jax import lax
from jax.experimental import pallas as pl
from jax.experimental.pallas import tpu as pltpu
from jax.experimental.pallas import tpu_sc as plsc

D_MODEL = 64
SCALE = math.sqrt(D_MODEL)
CHUNK = 128


@functools.cache
def _build_gather(B, V):
    info = plsc.get_sparse_core_info()
    nc, ns = info.num_cores, info.num_subcores
    nw = nc * ns
    b_per_w = B // nw
    n_chunks = b_per_w // CHUNK
    assert n_chunks % 2 == 0 and n_chunks >= 4
    mesh = plsc.VectorSubcoreMesh(core_axis_name="c", subcore_axis_name="s")

    @functools.partial(
        pl.kernel,
        mesh=mesh,
        compiler_params=pltpu.CompilerParams(use_tc_tiling_on_sc=False),
        out_type=jax.ShapeDtypeStruct((B, D_MODEL), jnp.float32),
        scratch_types=[
            pltpu.VMEM((n_chunks, CHUNK), jnp.int32),
            pltpu.VMEM((CHUNK, D_MODEL), jnp.float32),
            pltpu.VMEM((CHUNK, D_MODEL), jnp.float32),
            pltpu.SemaphoreType.DMA,
            pltpu.SemaphoreType.DMA,
            pltpu.SemaphoreType.DMA,
            pltpu.SemaphoreType.DMA,
        ],
    )
    def emb_kernel(idx_hbm, lut_hbm, out_hbm, idx_v, buf0, buf1,
                   gsem0, gsem1, wsem0, wsem1):
        wid = lax.axis_index("s") * nc + lax.axis_index("c")
        base = wid * b_per_w
        pltpu.sync_copy(idx_hbm.at[pl.ds(wid * n_chunks, n_chunks)], idx_v)

        bufs = (buf0, buf1)
        gsems = (gsem0, gsem1)
        wsems = (wsem0, wsem1)

        def gather_start(c, b):
            pltpu.async_copy(lut_hbm.at[idx_v.at[c]], bufs[b], gsems[b])

        def gather_wait(b):
            pltpu.make_async_copy(lut_hbm.at[idx_v.at[0]], bufs[b],
                                  gsems[b]).wait()

        def scale_buf(b):
            buf = bufs[b]

            def rbody(r8, carry):
                for rr in range(8):
                    r = r8 * 8 + rr
                    for j in range(4):
                        sl = pl.ds(16 * j, 16)
                        buf[r, sl] = buf[r, sl] * SCALE
                return carry

            lax.fori_loop(0, CHUNK // 8, rbody, 0)

        def write_start(c, b):
            pltpu.async_copy(bufs[b],
                             out_hbm.at[pl.ds(base + c * CHUNK, CHUNK)],
                             wsems[b])

        def write_wait(b):
            pltpu.make_async_copy(bufs[b], out_hbm.at[pl.ds(0, CHUNK)],
                                  wsems[b]).wait()

        gather_start(0, 0)
        gather_start(1, 1)

        def body(t, carry):
            c = 2 * t
            gather_wait(0)
            scale_buf(0)
            write_start(c, 0)
            gather_wait(1)
            scale_buf(1)
            write_start(c + 1, 1)
            write_wait(0)
            gather_start(c + 2, 0)
            write_wait(1)
            gather_start(c + 3, 1)
            return carry

        lax.fori_loop(0, n_chunks // 2 - 1, body, 0)

        c = n_chunks - 2
        gather_wait(0)
        scale_buf(0)
        write_start(c, 0)
        gather_wait(1)
        scale_buf(1)
        write_start(c + 1, 1)
        write_wait(0)
        write_wait(1)

    return emb_kernel


def kernel(x, lut):
    R, C = x.shape
    B = R * C
    xi = x.reshape(-1).astype(jnp.int32)
    idx2 = xi.reshape(B // CHUNK, CHUNK)
    out = _build_gather(B, lut.shape[0])(idx2, lut)
    return out.reshape(R, C, D_MODEL)


# speedup vs baseline: 1.1557x; 1.0443x over previous
import functools
import math

import jax
import jax.numpy as jnp
from jax import lax
from jax.experimental import pallas as pl
from jax.experimental.pallas import tpu as pltpu
from jax.experimental.pallas import tpu_sc as plsc

D_MODEL = 64
SCALE = math.sqrt(D_MODEL)
CHUNK = 128
NBUF = 4


@functools.cache
def _build_gather(B, V):
    info = plsc.get_sparse_core_info()
    nc, ns = info.num_cores, info.num_subcores
    nw = nc * ns
    b_per_w = B // nw
    n_chunks = b_per_w // CHUNK
    assert n_chunks % NBUF == 0 and n_chunks >= 2 * NBUF
    mesh = plsc.VectorSubcoreMesh(core_axis_name="c", subcore_axis_name="s")

    @functools.partial(
        pl.kernel,
        mesh=mesh,
        compiler_params=pltpu.CompilerParams(use_tc_tiling_on_sc=False),
        out_type=jax.ShapeDtypeStruct((B, D_MODEL), jnp.float32),
        scratch_types=[
            pltpu.VMEM((n_chunks, CHUNK), jnp.int32),
            pltpu.VMEM((CHUNK, D_MODEL), jnp.float32),
            pltpu.VMEM((CHUNK, D_MODEL), jnp.float32),
            pltpu.VMEM((CHUNK, D_MODEL), jnp.float32),
            pltpu.VMEM((CHUNK, D_MODEL), jnp.float32),
            pltpu.SemaphoreType.DMA,
            pltpu.SemaphoreType.DMA,
            pltpu.SemaphoreType.DMA,
            pltpu.SemaphoreType.DMA,
            pltpu.SemaphoreType.DMA,
            pltpu.SemaphoreType.DMA,
            pltpu.SemaphoreType.DMA,
            pltpu.SemaphoreType.DMA,
        ],
    )
    def emb_kernel(idx_hbm, lut_hbm, out_hbm, idx_v, buf0, buf1, buf2, buf3,
                   gsem0, gsem1, gsem2, gsem3, wsem0, wsem1, wsem2, wsem3):
        wid = lax.axis_index("s") * nc + lax.axis_index("c")
        base = wid * b_per_w
        pltpu.sync_copy(idx_hbm.at[pl.ds(wid * n_chunks, n_chunks)], idx_v)

        bufs = (buf0, buf1, buf2, buf3)
        gsems = (gsem0, gsem1, gsem2, gsem3)
        wsems = (wsem0, wsem1, wsem2, wsem3)

        def gather_start(c, b):
            pltpu.async_copy(lut_hbm.at[idx_v.at[c]], bufs[b], gsems[b])

        def gather_wait(b):
            pltpu.make_async_copy(lut_hbm.at[idx_v.at[0]], bufs[b],
                                  gsems[b]).wait()

        def scale_buf(b):
            buf = bufs[b]

            def rbody(r8, carry):
                for rr in range(8):
                    r = r8 * 8 + rr
                    for j in range(4):
                        sl = pl.ds(16 * j, 16)
                        buf[r, sl] = buf[r, sl] * SCALE
                return carry

            lax.fori_loop(0, CHUNK // 8, rbody, 0)

        def write_start(c, b):
            pltpu.async_copy(bufs[b],
                             out_hbm.at[pl.ds(base + c * CHUNK, CHUNK)],
                             wsems[b])

        def write_wait(b):
            pltpu.make_async_copy(bufs[b], out_hbm.at[pl.ds(0, CHUNK)],
                                  wsems[b]).wait()

        for b in range(NBUF):
            gather_start(b, b)

        def body(t, carry):
            c0 = NBUF * t
            for k in range(NBUF):
                c = c0 + k
                gather_wait(k)
                scale_buf(k)
                write_start(c, k)
                write_wait(k)
                gather_start(c + NBUF, k)
            return carry

        lax.fori_loop(0, n_chunks // NBUF - 1, body, 0)

        c0 = n_chunks - NBUF
        for k in range(NBUF):
            gather_wait(k)
            scale_buf(k)
            write_start(c0 + k, k)
        for k in range(NBUF):
            write_wait(k)

    return emb_kernel


def kernel(x, lut):
    R, C = x.shape
    B = R * C
    xi = x.reshape(-1).astype(jnp.int32)
    idx2 = xi.reshape(B // CHUNK, CHUNK)
    out = _build_gather(B, lut.shape[0])(idx2, lut)
    return out.reshape(R, C, D_MODEL)
